# Initial kernel scaffold; baseline (speedup 1.0000x reference)
import jax
import jax.numpy as jnp
from jax.experimental import pallas as pl


def kernel(input, agents_x, agents_y, agents_w, samples_x, samples_y, samples_pi):
    raise NotImplementedError("write your pallas kernel here")


# speedup vs baseline: 3.8881x; 3.8881x over previous
import jax
import jax.numpy as jnp
from jax.experimental import pallas as pl

_BN = 256


def _reward_kernel(x_ref, w_ref, o_ref):
    bn = x_ref.shape[0]
    x = x_ref[...].reshape(bn * x_ref.shape[1], x_ref.shape[2])
    s = jax.lax.dot_general(
        x, w_ref[...], (((1,), (0,)), ((), ())),
        preferred_element_type=jnp.float32,
    )
    s_x = s[:, 0]
    s_y = s[:, 1]
    w_s = s[:, 2]
    s_kx = s[:, 3]
    s_ky = s[:, 4]
    pi_k = s[:, 5]
    y = jnp.abs(
        pi_k * (
            jnp.log(pi_k / w_s)
            + 0.5 * (
                jnp.log(s_x * s_y / (s_kx * s_ky))
                + (s_kx * s_y + s_x * s_ky) / (s_x * s_y)
                - 2.0
            )
        )
    )
    o_ref[...] = y.reshape(bn, x_ref.shape[1])


def kernel(input, agents_x, agents_y, agents_w, samples_x, samples_y, samples_pi):
    n, a, m = input.shape
    w = jnp.stack(
        [agents_x, agents_y, agents_w, samples_x, samples_y, samples_pi], axis=1
    )
    return pl.pallas_call(
        _reward_kernel,
        grid=(n // _BN,),
        in_specs=[
            pl.BlockSpec((_BN, a, m), lambda i: (i, 0, 0)),
            pl.BlockSpec((m, 6), lambda i: (0, 0)),
        ],
        out_specs=pl.BlockSpec((_BN, a), lambda i: (i, 0)),
        out_shape=jax.ShapeDtypeStruct((n, a), jnp.float32),
    )(input, w)


# speedup vs baseline: 5.4617x; 1.4047x over previous
import jax
import jax.numpy as jnp
from jax.experimental import pallas as pl

_BR = 2048


def _reward_kernel(x_ref, w_ref, o_ref):
    x = x_ref[...]
    s = jax.lax.dot_general(
        w_ref[...], x, (((1,), (1,)), ((), ())),
        preferred_element_type=jnp.float32,
    )
    s_x = s[0:1]
    s_y = s[1:2]
    w_s = s[2:3]
    s_kx = s[3:4]
    s_ky = s[4:5]
    pi_k = s[5:6]
    y = jnp.abs(
        pi_k * (
            jnp.log(pi_k / w_s)
            + 0.5 * (
                jnp.log(s_x * s_y / (s_kx * s_ky))
                + (s_kx * s_y + s_x * s_ky) / (s_x * s_y)
                - 2.0
            )
        )
    )
    o_ref[...] = y.reshape(1, 1, y.shape[1])


def kernel(input, agents_x, agents_y, agents_w, samples_x, samples_y, samples_pi):
    n, a, m = input.shape
    rows = n * a
    x2 = input.reshape(rows, m)
    wt = jnp.stack(
        [agents_x, agents_y, agents_w, samples_x, samples_y, samples_pi], axis=0
    )
    nblk = rows // _BR
    out = pl.pallas_call(
        _reward_kernel,
        grid=(nblk,),
        in_specs=[
            pl.BlockSpec((_BR, m), lambda i: (i, 0)),
            pl.BlockSpec((6, m), lambda i: (0, 0)),
        ],
        out_specs=pl.BlockSpec((1, 1, _BR), lambda i: (i, 0, 0)),
        out_shape=jax.ShapeDtypeStruct((nblk, 1, _BR), jnp.float32),
    )(x2, wt)
    return out.reshape(n, a)
